# Initial kernel scaffold; baseline (speedup 1.0000x reference)
import jax
import jax.numpy as jnp
from jax.experimental import pallas as pl


def kernel(trees, indexes, mask_padding, W_emb, b_emb, Wa, ba, v, W1, b1, W2, b2):
    raise NotImplementedError("write your pallas kernel here")


# speedup vs baseline: 1.4878x; 1.4878x over previous
import jax
import jax.numpy as jnp
from jax.experimental import pallas as pl
from jax.experimental.pallas import tpu as pltpu

_P, _N, _F, _H = 64, 1024, 512, 512


def _plan_body(trees_ref, mask_ref, Wemb_ref, bemb_ref, Wa_ref, ba_ref, v_ref,
               W1_ref, b1_ref, W2_ref, b2_ref, out_ref, comb_ref):
    t = trees_ref[0]
    emb = jnp.dot(Wemb_ref[...], t, preferred_element_type=jnp.float32)
    emb = jnp.maximum(emb + bemb_ref[...], 0.0)
    a = jnp.dot(Wa_ref[...], emb, preferred_element_type=jnp.float32)
    a = jnp.tanh(a + ba_ref[...])
    scores = jnp.dot(v_ref[...], a, preferred_element_type=jnp.float32)
    scores = jnp.where(mask_ref[0] > 0.5, -1e9, scores)
    m = jnp.max(scores, axis=1, keepdims=True)
    e = jnp.exp(scores - m)
    attn = e / jnp.sum(e, axis=1, keepdims=True)
    pool = jax.lax.dot_general(emb, attn, (((1,), (1,)), ((), ())),
                               preferred_element_type=jnp.float32)
    root = emb[:, 1:2]
    comb = jnp.concatenate([root, pool], axis=0)
    hid = jax.lax.dot_general(W1_ref[...], comb, (((1,), (0,)), ((), ())),
                              preferred_element_type=jnp.float32)
    hid = jnp.maximum(hid + b1_ref[...], 0.0)
    out = jnp.dot(W2_ref[...], hid, preferred_element_type=jnp.float32)
    out_ref[...] = (out + b2_ref[...]).reshape(1, 1, 1)
    comb_ref[...] = comb.T.reshape(1, 1, 2 * _H)


@jax.jit
def _run(trees, mask_f, W_emb, b_emb_c, Wa, ba_c, v_row, W1, b1_c, W2_row, b2_c):
    return pl.pallas_call(
        _plan_body,
        grid=(_P,),
        in_specs=[
            pl.BlockSpec((1, _F, _N), lambda p: (p, 0, 0)),
            pl.BlockSpec((1, 1, _N), lambda p: (p, 0, 0)),
            pl.BlockSpec((_H, _F), lambda p: (0, 0)),
            pl.BlockSpec((_H, 1), lambda p: (0, 0)),
            pl.BlockSpec((_H, _H), lambda p: (0, 0)),
            pl.BlockSpec((_H, 1), lambda p: (0, 0)),
            pl.BlockSpec((1, _H), lambda p: (0, 0)),
            pl.BlockSpec((_H, 2 * _H), lambda p: (0, 0)),
            pl.BlockSpec((_H, 1), lambda p: (0, 0)),
            pl.BlockSpec((1, _H), lambda p: (0, 0)),
            pl.BlockSpec((1, 1), lambda p: (0, 0)),
        ],
        out_specs=[
            pl.BlockSpec((1, 1, 1), lambda p: (p, 0, 0)),
            pl.BlockSpec((1, 1, 2 * _H), lambda p: (p, 0, 0)),
        ],
        out_shape=[
            jax.ShapeDtypeStruct((_P, 1, 1), jnp.float32),
            jax.ShapeDtypeStruct((_P, 1, 2 * _H), jnp.float32),
        ],
        compiler_params=pltpu.CompilerParams(
            dimension_semantics=("arbitrary",)),
    )(trees, mask_f, W_emb, b_emb_c, Wa, ba_c, v_row, W1, b1_c, W2_row, b2_c)


def kernel(trees, indexes, mask_padding, W_emb, b_emb, Wa, ba, v, W1, b1, W2, b2):
    del indexes
    mask_f = mask_padding.astype(jnp.float32).reshape(_P, 1, _N)
    out, combined = _run(
        trees, mask_f, W_emb, b_emb.reshape(_H, 1), Wa, ba.reshape(_H, 1),
        v.reshape(1, _H), W1, b1.reshape(_H, 1), W2.reshape(1, _H),
        b2.reshape(1, 1))
    return (out.reshape(_P, 1), combined.reshape(_P, 2 * _H))


# speedup vs baseline: 2.1136x; 1.4206x over previous
import jax
import jax.numpy as jnp
from jax.experimental import pallas as pl
from jax.experimental.pallas import tpu as pltpu

_P, _N, _F, _H = 64, 1024, 512, 512
_PB = 2
_STEPS = _P // _PB


def _body(trees_ref, mask_ref, Wemb_ref, bemb_ref, Wa_ref, ba_ref, v_ref,
          W1_ref, b1_ref, W2_ref, b2_ref, out_ref, comb_ref):
    i = pl.program_id(0)
    for j in range(_PB):
        t = trees_ref[j]
        emb = jnp.dot(Wemb_ref[...], t, preferred_element_type=jnp.float32)
        emb = jnp.maximum(emb + bemb_ref[...], 0.0)
        a = jnp.dot(Wa_ref[...], emb, preferred_element_type=jnp.float32)
        a = jnp.tanh(a + ba_ref[...])
        scores = jnp.dot(v_ref[...], a, preferred_element_type=jnp.float32)
        scores = jnp.where(mask_ref[j] > 0.5, -1e9, scores)
        m = jnp.max(scores, axis=1, keepdims=True)
        e = jnp.exp(scores - m)
        attn = e / jnp.sum(e, axis=1, keepdims=True)
        pool = jax.lax.dot_general(emb, attn, (((1,), (1,)), ((), ())),
                                   preferred_element_type=jnp.float32)
        root = emb[:, 1:2]
        comb = jnp.concatenate([root, pool], axis=0)
        comb_ref[pl.ds(i * _PB + j, 1), :] = comb.T

    @pl.when(i == _STEPS - 1)
    def _tail():
        c = comb_ref[...]
        hid = jnp.dot(c, W1_ref[...], preferred_element_type=jnp.float32)
        hid = jnp.maximum(hid + b1_ref[...], 0.0)
        out = jnp.dot(hid, W2_ref[...], preferred_element_type=jnp.float32)
        out_ref[...] = out + b2_ref[...]


@jax.jit
def _run(trees, mask_f, W_emb, b_emb_c, Wa, ba_c, v_row, W1, b1_r, W2_row, b2_c):
    return pl.pallas_call(
        _body,
        grid=(_STEPS,),
        in_specs=[
            pl.BlockSpec((_PB, _F, _N), lambda i: (i, 0, 0)),
            pl.BlockSpec((_PB, 1, _N), lambda i: (i, 0, 0)),
            pl.BlockSpec((_H, _F), lambda i: (0, 0)),
            pl.BlockSpec((_H, 1), lambda i: (0, 0)),
            pl.BlockSpec((_H, _H), lambda i: (0, 0)),
            pl.BlockSpec((_H, 1), lambda i: (0, 0)),
            pl.BlockSpec((1, _H), lambda i: (0, 0)),
            pl.BlockSpec((2 * _H, _H), lambda i: (0, 0)),
            pl.BlockSpec((1, _H), lambda i: (0, 0)),
            pl.BlockSpec((_H, 1), lambda i: (0, 0)),
            pl.BlockSpec((1, 1), lambda i: (0, 0)),
        ],
        out_specs=[
            pl.BlockSpec((_P, 1), lambda i: (0, 0)),
            pl.BlockSpec((_P, 2 * _H), lambda i: (0, 0)),
        ],
        out_shape=[
            jax.ShapeDtypeStruct((_P, 1), jnp.float32),
            jax.ShapeDtypeStruct((_P, 2 * _H), jnp.float32),
        ],
        compiler_params=pltpu.CompilerParams(
            dimension_semantics=("arbitrary",)),
    )(trees, mask_f, W_emb, b_emb_c, Wa, ba_c, v_row, W1, b1_r, W2_row, b2_c)


def kernel(trees, indexes, mask_padding, W_emb, b_emb, Wa, ba, v, W1, b1, W2, b2):
    del indexes
    mask_f = mask_padding.astype(jnp.float32).reshape(_P, 1, _N)
    out, combined = _run(
        trees, mask_f, W_emb, b_emb.reshape(_H, 1), Wa, ba.reshape(_H, 1),
        v.reshape(1, _H), W1.T, b1.reshape(1, _H), W2.reshape(_H, 1),
        b2.reshape(1, 1))
    return (out, combined)


# speedup vs baseline: 2.1857x; 1.0341x over previous
import jax
import jax.numpy as jnp
from jax.experimental import pallas as pl
from jax.experimental.pallas import tpu as pltpu

_P, _N, _F, _H = 64, 1024, 512, 512
_PB = 4
_STEPS = _P // _PB


def _body(trees_ref, mask_ref, Wemb_ref, bemb_ref, Wa_ref, ba_ref, v_ref,
          W1_ref, b1_ref, W2_ref, b2_ref, out_ref, comb_ref):
    i = pl.program_id(0)
    for j in range(_PB):
        t = trees_ref[j]
        emb = jnp.dot(Wemb_ref[...], t, preferred_element_type=jnp.float32)
        emb = jnp.maximum(emb + bemb_ref[...], 0.0)
        a = jnp.dot(Wa_ref[...], emb, preferred_element_type=jnp.float32)
        a = jnp.tanh(a + ba_ref[...])
        scores = jnp.dot(v_ref[...], a, preferred_element_type=jnp.float32)
        scores = jnp.where(mask_ref[j] > 0.5, -1e9, scores)
        m = jnp.max(scores, axis=1, keepdims=True)
        e = jnp.exp(scores - m)
        attn = e / jnp.sum(e, axis=1, keepdims=True)
        pool = jax.lax.dot_general(emb, attn, (((1,), (1,)), ((), ())),
                                   preferred_element_type=jnp.float32)
        root = emb[:, 1:2]
        comb = jnp.concatenate([root, pool], axis=0)
        comb_ref[pl.ds(i * _PB + j, 1), :] = comb.T

    @pl.when(i == _STEPS - 1)
    def _tail():
        c = comb_ref[...]
        hid = jnp.dot(c, W1_ref[...], preferred_element_type=jnp.float32)
        hid = jnp.maximum(hid + b1_ref[...], 0.0)
        out = jnp.dot(hid, W2_ref[...], preferred_element_type=jnp.float32)
        out_ref[...] = out + b2_ref[...]


@jax.jit
def _run(trees, mask_f, W_emb, b_emb_c, Wa, ba_c, v_row, W1, b1_r, W2_row, b2_c):
    return pl.pallas_call(
        _body,
        grid=(_STEPS,),
        in_specs=[
            pl.BlockSpec((_PB, _F, _N), lambda i: (i, 0, 0)),
            pl.BlockSpec((_PB, 1, _N), lambda i: (i, 0, 0)),
            pl.BlockSpec((_H, _F), lambda i: (0, 0)),
            pl.BlockSpec((_H, 1), lambda i: (0, 0)),
            pl.BlockSpec((_H, _H), lambda i: (0, 0)),
            pl.BlockSpec((_H, 1), lambda i: (0, 0)),
            pl.BlockSpec((1, _H), lambda i: (0, 0)),
            pl.BlockSpec((2 * _H, _H), lambda i: (0, 0)),
            pl.BlockSpec((1, _H), lambda i: (0, 0)),
            pl.BlockSpec((_H, 1), lambda i: (0, 0)),
            pl.BlockSpec((1, 1), lambda i: (0, 0)),
        ],
        out_specs=[
            pl.BlockSpec((_P, 1), lambda i: (0, 0)),
            pl.BlockSpec((_P, 2 * _H), lambda i: (0, 0)),
        ],
        out_shape=[
            jax.ShapeDtypeStruct((_P, 1), jnp.float32),
            jax.ShapeDtypeStruct((_P, 2 * _H), jnp.float32),
        ],
        compiler_params=pltpu.CompilerParams(
            dimension_semantics=("arbitrary",)),
    )(trees, mask_f, W_emb, b_emb_c, Wa, ba_c, v_row, W1, b1_r, W2_row, b2_c)


def kernel(trees, indexes, mask_padding, W_emb, b_emb, Wa, ba, v, W1, b1, W2, b2):
    del indexes
    mask_f = mask_padding.astype(jnp.float32).reshape(_P, 1, _N)
    out, combined = _run(
        trees, mask_f, W_emb, b_emb.reshape(_H, 1), Wa, ba.reshape(_H, 1),
        v.reshape(1, _H), W1.T, b1.reshape(1, _H), W2.reshape(_H, 1),
        b2.reshape(1, 1))
    return (out, combined)


# speedup vs baseline: 2.1926x; 1.0032x over previous
import jax
import jax.numpy as jnp
from jax.experimental import pallas as pl
from jax.experimental.pallas import tpu as pltpu

_P, _N, _F, _H = 64, 1024, 512, 512
_PB = 8
_STEPS = _P // _PB


def _body(trees_ref, mask_ref, Wemb_ref, bemb_ref, Wa_ref, ba_ref, v_ref,
          W1_ref, b1_ref, W2_ref, b2_ref, out_ref, comb_ref):
    i = pl.program_id(0)
    for j in range(_PB):
        t = trees_ref[j]
        emb = jnp.dot(Wemb_ref[...], t, preferred_element_type=jnp.float32)
        emb = jnp.maximum(emb + bemb_ref[...], 0.0)
        a = jnp.dot(Wa_ref[...], emb, preferred_element_type=jnp.float32)
        a = jnp.tanh(a + ba_ref[...])
        scores = jnp.dot(v_ref[...], a, preferred_element_type=jnp.float32)
        scores = jnp.where(mask_ref[j] > 0.5, -1e9, scores)
        m = jnp.max(scores, axis=1, keepdims=True)
        e = jnp.exp(scores - m)
        attn = e / jnp.sum(e, axis=1, keepdims=True)
        pool = jax.lax.dot_general(emb, attn, (((1,), (1,)), ((), ())),
                                   preferred_element_type=jnp.float32)
        root = emb[:, 1:2]
        comb = jnp.concatenate([root, pool], axis=0)
        comb_ref[pl.ds(i * _PB + j, 1), :] = comb.T

    @pl.when(i == _STEPS - 1)
    def _tail():
        c = comb_ref[...]
        hid = jnp.dot(c, W1_ref[...], preferred_element_type=jnp.float32)
        hid = jnp.maximum(hid + b1_ref[...], 0.0)
        out = jnp.dot(hid, W2_ref[...], preferred_element_type=jnp.float32)
        out_ref[...] = out + b2_ref[...]


@jax.jit
def _run(trees, mask_f, W_emb, b_emb_c, Wa, ba_c, v_row, W1, b1_r, W2_row, b2_c):
    return pl.pallas_call(
        _body,
        grid=(_STEPS,),
        in_specs=[
            pl.BlockSpec((_PB, _F, _N), lambda i: (i, 0, 0)),
            pl.BlockSpec((_PB, 1, _N), lambda i: (i, 0, 0)),
            pl.BlockSpec((_H, _F), lambda i: (0, 0)),
            pl.BlockSpec((_H, 1), lambda i: (0, 0)),
            pl.BlockSpec((_H, _H), lambda i: (0, 0)),
            pl.BlockSpec((_H, 1), lambda i: (0, 0)),
            pl.BlockSpec((1, _H), lambda i: (0, 0)),
            pl.BlockSpec((2 * _H, _H), lambda i: (0, 0)),
            pl.BlockSpec((1, _H), lambda i: (0, 0)),
            pl.BlockSpec((_H, 1), lambda i: (0, 0)),
            pl.BlockSpec((1, 1), lambda i: (0, 0)),
        ],
        out_specs=[
            pl.BlockSpec((_P, 1), lambda i: (0, 0)),
            pl.BlockSpec((_P, 2 * _H), lambda i: (0, 0)),
        ],
        out_shape=[
            jax.ShapeDtypeStruct((_P, 1), jnp.float32),
            jax.ShapeDtypeStruct((_P, 2 * _H), jnp.float32),
        ],
        compiler_params=pltpu.CompilerParams(
            dimension_semantics=("arbitrary",)),
    )(trees, mask_f, W_emb, b_emb_c, Wa, ba_c, v_row, W1, b1_r, W2_row, b2_c)


def kernel(trees, indexes, mask_padding, W_emb, b_emb, Wa, ba, v, W1, b1, W2, b2):
    del indexes
    mask_f = mask_padding.astype(jnp.float32).reshape(_P, 1, _N)
    out, combined = _run(
        trees, mask_f, W_emb, b_emb.reshape(_H, 1), Wa, ba.reshape(_H, 1),
        v.reshape(1, _H), W1.T, b1.reshape(1, _H), W2.reshape(_H, 1),
        b2.reshape(1, 1))
    return (out, combined)
